# baseline (speedup 1.0000x reference)
import functools

import jax
import jax.numpy as jnp
from jax import lax
from jax.experimental import pallas as pl
from jax.experimental.pallas import tpu as pltpu
from jax.experimental.pallas import tpu_sc as plsc

_B = 128
_S = 197
_LP = 20
_D = 768
_NPOOL = 30
_PROW = _LP * _D
_XROW = _S * _D
_OROW = _PROW + _XROW
_NC = 2
_NS = 16
_NW = _NC * _NS
_HCOL = _XROW // 2
_CW = 4096
_NFULL = _HCOL // _CW
_TAIL = _HCOL - _NFULL * _CW
_NT = _NFULL + 1
_NBUF = 3
_LEAD = 2


def _chunk(c):
    if c < _NFULL:
        return c * _CW, _CW
    return _NFULL * _CW, _TAIL


def _sc_concat(x2, idx3, p2):
    mesh = plsc.VectorSubcoreMesh(core_axis_name="c", subcore_axis_name="s")

    @functools.partial(
        pl.kernel,
        mesh=mesh,
        out_type=jax.ShapeDtypeStruct((_B, _OROW), jnp.float32),
        scratch_types=[
            pltpu.VMEM((2, 2), jnp.int32),
            pltpu.VMEM((2, _PROW), jnp.float32),
            pltpu.VMEM((_NBUF, 8, _CW), jnp.float32),
            pltpu.SemaphoreType.DMA,
            pltpu.SemaphoreType.DMA,
            pltpu.SemaphoreType.DMA,
            pltpu.SemaphoreType.DMA,
            pltpu.SemaphoreType.DMA,
            pltpu.SemaphoreType.DMA,
            pltpu.SemaphoreType.DMA,
        ],
    )
    def body(x_hbm, idx_hbm, p_hbm, out_hbm, idx_v, pbuf, xbuf,
             s_r0, s_r1, s_r2, s_w0, s_w1, s_w2, s_p):
        sem_r = (s_r0, s_r1, s_r2)
        sem_w = (s_w0, s_w1, s_w2)
        wid = lax.axis_index("s") * _NC + lax.axis_index("c")
        grp = wid // 2
        half = lax.rem(wid, 2)
        row0 = pl.multiple_of(grp * 8, 8)
        col0 = half * _HCOL

        def start_read(c, slot):
            o, w = _chunk(c)
            return pltpu.async_copy(
                x_hbm.at[pl.ds(row0, 8), pl.ds(col0 + o, w)],
                xbuf.at[slot, :, pl.ds(0, w)],
                sem_r[slot],
            )

        def start_write(c, slot):
            o, w = _chunk(c)
            return pltpu.async_copy(
                xbuf.at[slot, :, pl.ds(0, w)],
                out_hbm.at[pl.ds(row0, 8), pl.ds(_PROW + col0 + o, w)],
                sem_w[slot],
            )

        reads, writes = {}, {}
        for t in range(_LEAD):
            reads[t] = start_read(t, t % _NBUF)

        pltpu.sync_copy(idx_hbm.at[wid], idx_v)
        for hh in range(2):
            pltpu.async_copy(p_hbm.at[idx_v.at[hh]], pbuf, s_p).wait()
            for ii in range(2):
                pltpu.sync_copy(
                    pbuf.at[ii],
                    out_hbm.at[wid * 4 + hh * 2 + ii, pl.ds(0, _PROW)],
                )

        for t in range(_LEAD, _NT + _LEAD):
            if t < _NT:
                slot = t % _NBUF
                if t >= _NBUF:
                    writes.pop(t - _NBUF).wait()
                reads[t] = start_read(t, slot)
            cw = t - _LEAD
            slot = cw % _NBUF
            reads.pop(cw).wait()
            writes[cw] = start_write(cw, slot)

        for c in sorted(writes):
            writes.pop(c).wait()

    return body(x2, idx3, p2)


def kernel(x, prompt_idx, prompts):
    x2 = x.reshape(_B, _XROW)
    idx3 = prompt_idx.astype(jnp.int32).reshape(_NW, 2, 2)
    p2 = prompts.reshape(_NPOOL, _PROW)
    out = _sc_concat(x2, idx3, p2)
    return out.reshape(_B, _LP + _S, _D)


# speedup vs baseline: 1.7600x; 1.7600x over previous
import jax
import jax.numpy as jnp
from jax.experimental import pallas as pl
from jax.experimental.pallas import tpu as pltpu

_B = 128
_S = 197
_LP = 20
_D = 768
_BB = 8


def _body(idx_ref, p_ref, x_ref, out_ref):
    g = pl.program_id(0)
    out_ref[:, _LP:, :] = x_ref[...]
    for i in range(_BB):
        out_ref[i, :_LP, :] = p_ref[idx_ref[g * _BB + i]]


def kernel(x, prompt_idx, prompts):
    idx = prompt_idx.astype(jnp.int32)
    n_pool, lp, d = prompts.shape
    grid_spec = pltpu.PrefetchScalarGridSpec(
        num_scalar_prefetch=1,
        grid=(_B // _BB,),
        in_specs=[
            pl.BlockSpec((n_pool, lp, d), lambda b, idx_ref: (0, 0, 0)),
            pl.BlockSpec((_BB, _S, _D), lambda b, idx_ref: (b, 0, 0)),
        ],
        out_specs=pl.BlockSpec((_BB, _LP + _S, _D), lambda b, idx_ref: (b, 0, 0)),
    )
    out = pl.pallas_call(
        _body,
        grid_spec=grid_spec,
        out_shape=jax.ShapeDtypeStruct((_B, _LP + _S, _D), jnp.float32),
    )(idx, prompts, x)
    return out
